# baseline (speedup 1.0000x reference)
import functools

import jax
import jax.numpy as jnp
from jax import lax
from jax.experimental import pallas as pl
from jax.experimental.pallas import tpu as pltpu
from jax.experimental.pallas import tpu_sc as plsc

N = 50000
E = 800000
C = 32
FIN = 64
ED = 9
MD = 3

NC = 2
NS = 16
LANES = 16
NW = NC * NS

EK = 384
EPAD = ((E + NW * EK - 1) // (NW * EK)) * (NW * EK)
CHUNKS = EPAD // (NW * EK)
NACC = 50432
RPT = NACC // NS
ZC = 632
NZ = RPT // ZC
NACC1 = NACC
RPT1 = NACC1 // NS


def _prep_nodes_body(x_ref, wt_ref, avs_ref, avd_ref, xp_ref, as_ref, ad_ref):
    xp = jnp.dot(x_ref[...], wt_ref[...], preferred_element_type=jnp.float32)
    xp_ref[...] = xp
    as_ref[...] = jnp.sum(xp * avs_ref[...], axis=1, keepdims=True)
    ad_ref[...] = jnp.sum(xp * avd_ref[...], axis=1, keepdims=True)


def _finish_body(msg_ref, den_ref, sseg_ref, deg_ref, xp_ref, as_ref, ad_ref,
                 bias_ref, m0c_ref, out_ref):
    msg = msg_ref[0] + msg_ref[1]
    den = den_ref[0] + den_ref[1]
    sseg = sseg_ref[0] + sseg_ref[1]
    deg = deg_ref[0] + deg_ref[1]
    al = as_ref[...] + ad_ref[...] + sseg / jnp.maximum(deg, 1.0) + m0c_ref[0, 0]
    al = jnp.where(al >= 0.0, al, 0.2 * al)
    exl = jnp.exp(al)
    xp = xp_ref[...]
    out_ref[...] = (msg + xp * exl) / (den + exl + 1e-16) + bias_ref[...]


def _sc_body(ei_r, ea_r, mac_r, vem_r, as_r, ad_r, xp_r, z32_r, z1_r,
             msg_out, den_out, sseg_out, deg_out,
             accm, accd, accs, accg,
             srcv, dstv, eab, macb, asv, adv, exb, oneb, sbuf, msgv, vemv,
             sem0, sem1, sem2):
    cid = lax.axis_index("c")
    sid = lax.axis_index("s")
    wid = cid * NS + sid
    one16 = jnp.ones((LANES,), jnp.float32)
    iota = lax.iota(jnp.int32, LANES)
    pltpu.sync_copy(vem_r, vemv)
    vem = vemv[...]
    vcoef = [vem.at[jnp.full((LANES,), j, jnp.int32)].get(mode="promise_in_bounds")
             for j in range(ED + MD)]
    iota9 = iota * ED
    iota3 = iota * MD

    row0 = sid * RPT
    srow0 = sid * RPT1
    pltpu.sync_copy(z32_r, accm.at[pl.ds(row0, RPT)])
    pltpu.sync_copy(z1_r, accd.at[pl.ds(srow0, RPT1)])
    pltpu.sync_copy(z1_r, accs.at[pl.ds(srow0, RPT1)])
    pltpu.sync_copy(z1_r, accg.at[pl.ds(srow0, RPT1)])

    for gi in range(EK // LANES):
        oneb[pl.ds(gi * LANES, LANES)] = one16

    plsc.subcore_barrier()

    ebase = wid * (CHUNKS * EK)

    def chunk(j, carry):
        base = ebase + j * EK
        base_ea = jnp.minimum(base, E - EK)
        d9 = jnp.broadcast_to((base - base_ea) * ED, (LANES,)).astype(jnp.int32)
        d3 = jnp.broadcast_to((base - base_ea) * MD, (LANES,)).astype(jnp.int32)
        iota9d = iota9 + d9
        iota3d = iota3 + d3
        pltpu.sync_copy(ei_r.at[0, pl.ds(base, EK)], srcv)
        pltpu.sync_copy(ei_r.at[1, pl.ds(base, EK)], dstv)
        pltpu.sync_copy(ea_r.at[pl.ds(base_ea * ED, EK * ED)], eab)
        pltpu.sync_copy(mac_r.at[pl.ds(base_ea * MD, EK * MD)], macb)
        ca = pltpu.async_copy(as_r.at[srcv], asv, sem0)
        cb = pltpu.async_copy(ad_r.at[dstv], adv, sem1)
        cc = pltpu.async_copy(xp_r.at[srcv], msgv, sem2)
        ca.wait()
        cb.wait()
        cc.wait()
        for gi in range(EK // LANES):
            o = gi * LANES
            sg = jnp.zeros((LANES,), jnp.float32)
            for jj in range(ED):
                idx = jnp.minimum(iota9d + (o * ED + jj), EK * ED - 1)
                col = plsc.load_gather(eab, [idx])
                sg = sg + vcoef[jj] * col
            mg = jnp.zeros((LANES,), jnp.float32)
            for jj in range(MD):
                idx = jnp.minimum(iota3d + (o * MD + jj), EK * MD - 1)
                col = plsc.load_gather(macb, [idx])
                mg = mg + vcoef[ED + jj] * col
            sbuf[pl.ds(o, LANES)] = sg
            t = asv[pl.ds(o, LANES)] + adv[pl.ds(o, LANES)] + sg + mg
            t = jnp.where(t >= 0.0, t, 0.2 * t)
            exv = jnp.exp(t)
            exb[pl.ds(o, LANES)] = exv
            for l in range(LANES):
                e = o + l
                spl = exv.at[jnp.full((LANES,), l, jnp.int32)].get(
                    mode="promise_in_bounds")
                msgv[e, pl.ds(0, LANES)] = msgv[e, pl.ds(0, LANES)] * spl
                msgv[e, pl.ds(LANES, LANES)] = msgv[e, pl.ds(LANES, LANES)] * spl
        pltpu.sync_copy(msgv, accm.at[dstv], add=True)
        pltpu.sync_copy(exb, accd.at[dstv], add=True)
        pltpu.sync_copy(sbuf, accs.at[dstv], add=True)
        pltpu.sync_copy(oneb, accg.at[dstv], add=True)
        return carry

    lax.fori_loop(0, CHUNKS, chunk, 0)

    plsc.subcore_barrier()

    pltpu.sync_copy(accm.at[pl.ds(row0, RPT)],
                    msg_out.at[pl.ds(cid * NACC + row0, RPT)])
    pltpu.sync_copy(accd.at[pl.ds(srow0, RPT1)],
                    den_out.at[pl.ds(cid * NACC1 + srow0, RPT1)])
    pltpu.sync_copy(accs.at[pl.ds(srow0, RPT1)],
                    sseg_out.at[pl.ds(cid * NACC1 + srow0, RPT1)])
    pltpu.sync_copy(accg.at[pl.ds(srow0, RPT1)],
                    deg_out.at[pl.ds(cid * NACC1 + srow0, RPT1)])


_sc_edge_pass = functools.partial(
    pl.kernel,
    out_type=[
        jax.ShapeDtypeStruct((NC * NACC, C), jnp.float32),
        jax.ShapeDtypeStruct((NC * NACC1,), jnp.float32),
        jax.ShapeDtypeStruct((NC * NACC1,), jnp.float32),
        jax.ShapeDtypeStruct((NC * NACC1,), jnp.float32),
    ],
    mesh=plsc.VectorSubcoreMesh(core_axis_name="c", subcore_axis_name="s"),
    compiler_params=pltpu.CompilerParams(use_tc_tiling_on_sc=False,
                                         needs_layout_passes=False),
    scratch_types=[
        pltpu.VMEM_SHARED((NACC, C), jnp.float32),
        pltpu.VMEM_SHARED((NACC1,), jnp.float32),
        pltpu.VMEM_SHARED((NACC1,), jnp.float32),
        pltpu.VMEM_SHARED((NACC1,), jnp.float32),
        pltpu.VMEM((EK,), jnp.int32),
        pltpu.VMEM((EK,), jnp.int32),
        pltpu.VMEM((EK * ED,), jnp.float32),
        pltpu.VMEM((EK * MD,), jnp.float32),
        pltpu.VMEM((EK,), jnp.float32),
        pltpu.VMEM((EK,), jnp.float32),
        pltpu.VMEM((EK,), jnp.float32),
        pltpu.VMEM((EK,), jnp.float32),
        pltpu.VMEM((EK,), jnp.float32),
        pltpu.VMEM((EK, C), jnp.float32),
        pltpu.VMEM((LANES,), jnp.float32),
        pltpu.SemaphoreType.DMA,
        pltpu.SemaphoreType.DMA,
        pltpu.SemaphoreType.DMA,
    ],
)(_sc_body)


def kernel(x, edge_index, edge_attr, macro, W_src, att_src, att_dst,
           W_edge, att_edge, W_macro, att_macro, bias):
    f32 = jnp.float32
    avs = att_src[0, 0].astype(f32)
    avd = att_dst[0, 0].astype(f32)
    ve = (att_edge[0, 0] @ W_edge).astype(f32)
    vm = (att_macro[0, 0] @ W_macro).astype(f32)
    m0c = (macro[0] @ vm).reshape(1, 1).astype(f32)

    RB = 1000
    xp, a_src, a_dst = pl.pallas_call(
        _prep_nodes_body,
        grid=(N // RB,),
        in_specs=[
            pl.BlockSpec((RB, FIN), lambda i: (i, 0)),
            pl.BlockSpec((FIN, C), lambda i: (0, 0)),
            pl.BlockSpec((1, C), lambda i: (0, 0)),
            pl.BlockSpec((1, C), lambda i: (0, 0)),
        ],
        out_specs=[
            pl.BlockSpec((RB, C), lambda i: (i, 0)),
            pl.BlockSpec((RB, 1), lambda i: (i, 0)),
            pl.BlockSpec((RB, 1), lambda i: (i, 0)),
        ],
        out_shape=[
            jax.ShapeDtypeStruct((N, C), f32),
            jax.ShapeDtypeStruct((N, 1), f32),
            jax.ShapeDtypeStruct((N, 1), f32),
        ],
    )(x.astype(f32), W_src.T.astype(f32), avs.reshape(1, C), avd.reshape(1, C))

    npad = EPAD - E
    pad_rows = jnp.stack([jnp.zeros((npad,), jnp.int32),
                          jnp.full((npad,), N, jnp.int32)])
    ei_pad = jnp.concatenate([edge_index.astype(jnp.int32), pad_rows], axis=1)
    ea_flat = edge_attr.astype(f32).reshape(E * ED)
    mac_flat = macro.astype(f32).reshape(E * MD)
    vem = jnp.concatenate([ve, vm, jnp.zeros((LANES - ED - MD,), f32)])
    z32 = jnp.zeros((RPT, C), f32)
    z1 = jnp.zeros((RPT1,), f32)

    msg, den, sseg, deg = _sc_edge_pass(ei_pad, ea_flat, mac_flat, vem,
                                        a_src[:, 0], a_dst[:, 0], xp, z32, z1)
    msg = msg.reshape(NC, NACC, C)
    den = den.reshape(NC, NACC1, 1)
    sseg = sseg.reshape(NC, NACC1, 1)
    deg = deg.reshape(NC, NACC1, 1)

    out = pl.pallas_call(
        _finish_body,
        grid=(N // RB,),
        in_specs=[
            pl.BlockSpec((NC, RB, C), lambda i: (0, i, 0)),
            pl.BlockSpec((NC, RB, 1), lambda i: (0, i, 0)),
            pl.BlockSpec((NC, RB, 1), lambda i: (0, i, 0)),
            pl.BlockSpec((NC, RB, 1), lambda i: (0, i, 0)),
            pl.BlockSpec((RB, C), lambda i: (i, 0)),
            pl.BlockSpec((RB, 1), lambda i: (i, 0)),
            pl.BlockSpec((RB, 1), lambda i: (i, 0)),
            pl.BlockSpec((1, C), lambda i: (0, 0)),
            pl.BlockSpec((1, 1), lambda i: (0, 0)),
        ],
        out_specs=pl.BlockSpec((RB, C), lambda i: (i, 0)),
        out_shape=jax.ShapeDtypeStruct((N, C), f32),
    )(msg, den, sseg, deg, xp, a_src, a_dst,
      bias.reshape(1, C).astype(f32), m0c)
    return out


# speedup vs baseline: 2.0037x; 2.0037x over previous
import functools

import jax
import jax.numpy as jnp
from jax import lax
from jax.experimental import pallas as pl
from jax.experimental.pallas import tpu as pltpu
from jax.experimental.pallas import tpu_sc as plsc

N = 50000
E = 800000
C = 32
FIN = 64
ED = 9
MD = 3

NC = 2
NS = 16
LANES = 16
NW = NC * NS

EK = 512
EPAD = ((E + NW * EK - 1) // (NW * EK)) * (NW * EK)
CHUNKS = EPAD // (NW * EK)
NACC = 50432
RPT = NACC // NS
ZC = 632
NZ = RPT // ZC
NACC1 = NACC
RPT1 = NACC1 // NS


def _prep_nodes_body(x_ref, wt_ref, avs_ref, avd_ref, xp_ref, as_ref, ad_ref):
    xp = jnp.dot(x_ref[...], wt_ref[...], preferred_element_type=jnp.float32)
    xp_ref[...] = xp
    as_ref[...] = jnp.sum(xp * avs_ref[...], axis=1, keepdims=True)
    ad_ref[...] = jnp.sum(xp * avd_ref[...], axis=1, keepdims=True)


def _prep_edges_body(ea_ref, mac_ref, ve_ref, vm_ref, s_ref, m_ref):
    s_ref[...] = jnp.sum(ea_ref[...] * ve_ref[...], axis=1)
    m_ref[...] = jnp.sum(mac_ref[...] * vm_ref[...], axis=1)


def _finish_body(msg_ref, den_ref, sseg_ref, deg_ref, xp_ref, as_ref, ad_ref,
                 bias_ref, m0c_ref, out_ref):
    msg = msg_ref[0] + msg_ref[1]
    den = den_ref[0] + den_ref[1]
    sseg = sseg_ref[0] + sseg_ref[1]
    deg = deg_ref[0] + deg_ref[1]
    al = as_ref[...] + ad_ref[...] + sseg / jnp.maximum(deg, 1.0) + m0c_ref[0, 0]
    al = jnp.where(al >= 0.0, al, 0.2 * al)
    exl = jnp.exp(al)
    xp = xp_ref[...]
    out_ref[...] = (msg + xp * exl) / (den + exl + 1e-16) + bias_ref[...]


def _sc_body(src_r, dst_r, s_r, m_r, as_r, ad_r, xp_r, z32_r, z1_r,
             msg_out, den_out, sseg_out, deg_out,
             accm, accd, accs, accg,
             srcv, dstv, sv, mv, asv, adv, exb, oneb, msgv,
             sem0, sem1, sem2):
    cid = lax.axis_index("c")
    sid = lax.axis_index("s")
    wid = cid * NS + sid
    one16 = jnp.ones((LANES,), jnp.float32)

    row0 = sid * RPT
    srow0 = sid * RPT1
    pltpu.sync_copy(z32_r, accm.at[pl.ds(row0, RPT)])
    pltpu.sync_copy(z1_r, accd.at[pl.ds(srow0, RPT1)])
    pltpu.sync_copy(z1_r, accs.at[pl.ds(srow0, RPT1)])
    pltpu.sync_copy(z1_r, accg.at[pl.ds(srow0, RPT1)])

    for gi in range(EK // LANES):
        oneb[pl.ds(gi * LANES, LANES)] = one16

    plsc.subcore_barrier()

    ebase = wid * (CHUNKS * EK)

    def chunk(j, carry):
        base = ebase + j * EK
        pltpu.sync_copy(src_r.at[pl.ds(base, EK)], srcv)
        pltpu.sync_copy(dst_r.at[pl.ds(base, EK)], dstv)
        pltpu.sync_copy(s_r.at[pl.ds(base, EK)], sv)
        pltpu.sync_copy(m_r.at[pl.ds(base, EK)], mv)
        ca = pltpu.async_copy(as_r.at[srcv], asv, sem0)
        cb = pltpu.async_copy(ad_r.at[dstv], adv, sem1)
        cc = pltpu.async_copy(xp_r.at[srcv], msgv, sem2)
        ca.wait()
        cb.wait()
        cc.wait()
        for gi in range(EK // LANES):
            o = gi * LANES
            t = (asv[pl.ds(o, LANES)] + adv[pl.ds(o, LANES)]
                 + sv[pl.ds(o, LANES)] + mv[pl.ds(o, LANES)])
            t = jnp.where(t >= 0.0, t, 0.2 * t)
            exv = jnp.exp(t)
            exb[pl.ds(o, LANES)] = exv
            for l in range(LANES):
                e = o + l
                spl = exv.at[jnp.full((LANES,), l, jnp.int32)].get(
                    mode="promise_in_bounds")
                msgv[e, pl.ds(0, LANES)] = msgv[e, pl.ds(0, LANES)] * spl
                msgv[e, pl.ds(LANES, LANES)] = msgv[e, pl.ds(LANES, LANES)] * spl
        pltpu.sync_copy(msgv, accm.at[dstv], add=True)
        pltpu.sync_copy(exb, accd.at[dstv], add=True)
        pltpu.sync_copy(sv, accs.at[dstv], add=True)
        pltpu.sync_copy(oneb, accg.at[dstv], add=True)
        return carry

    lax.fori_loop(0, CHUNKS, chunk, 0)

    plsc.subcore_barrier()

    pltpu.sync_copy(accm.at[pl.ds(row0, RPT)],
                    msg_out.at[pl.ds(cid * NACC + row0, RPT)])
    pltpu.sync_copy(accd.at[pl.ds(srow0, RPT1)],
                    den_out.at[pl.ds(cid * NACC1 + srow0, RPT1)])
    pltpu.sync_copy(accs.at[pl.ds(srow0, RPT1)],
                    sseg_out.at[pl.ds(cid * NACC1 + srow0, RPT1)])
    pltpu.sync_copy(accg.at[pl.ds(srow0, RPT1)],
                    deg_out.at[pl.ds(cid * NACC1 + srow0, RPT1)])


_sc_edge_pass = functools.partial(
    pl.kernel,
    out_type=[
        jax.ShapeDtypeStruct((NC * NACC, C), jnp.float32),
        jax.ShapeDtypeStruct((NC * NACC1,), jnp.float32),
        jax.ShapeDtypeStruct((NC * NACC1,), jnp.float32),
        jax.ShapeDtypeStruct((NC * NACC1,), jnp.float32),
    ],
    mesh=plsc.VectorSubcoreMesh(core_axis_name="c", subcore_axis_name="s"),
    compiler_params=pltpu.CompilerParams(use_tc_tiling_on_sc=False),
    scratch_types=[
        pltpu.VMEM_SHARED((NACC, C), jnp.float32),
        pltpu.VMEM_SHARED((NACC1,), jnp.float32),
        pltpu.VMEM_SHARED((NACC1,), jnp.float32),
        pltpu.VMEM_SHARED((NACC1,), jnp.float32),
        pltpu.VMEM((EK,), jnp.int32),
        pltpu.VMEM((EK,), jnp.int32),
        pltpu.VMEM((EK,), jnp.float32),
        pltpu.VMEM((EK,), jnp.float32),
        pltpu.VMEM((EK,), jnp.float32),
        pltpu.VMEM((EK,), jnp.float32),
        pltpu.VMEM((EK,), jnp.float32),
        pltpu.VMEM((EK,), jnp.float32),
        pltpu.VMEM((EK, C), jnp.float32),
        pltpu.SemaphoreType.DMA,
        pltpu.SemaphoreType.DMA,
        pltpu.SemaphoreType.DMA,
    ],
)(_sc_body)


def kernel(x, edge_index, edge_attr, macro, W_src, att_src, att_dst,
           W_edge, att_edge, W_macro, att_macro, bias):
    f32 = jnp.float32
    avs = att_src[0, 0].astype(f32)
    avd = att_dst[0, 0].astype(f32)
    ve = (att_edge[0, 0] @ W_edge).astype(f32)
    vm = (att_macro[0, 0] @ W_macro).astype(f32)
    m0c = (macro[0] @ vm).reshape(1, 1).astype(f32)

    RB = 1000
    xp, a_src, a_dst = pl.pallas_call(
        _prep_nodes_body,
        grid=(N // RB,),
        in_specs=[
            pl.BlockSpec((RB, FIN), lambda i: (i, 0)),
            pl.BlockSpec((FIN, C), lambda i: (0, 0)),
            pl.BlockSpec((1, C), lambda i: (0, 0)),
            pl.BlockSpec((1, C), lambda i: (0, 0)),
        ],
        out_specs=[
            pl.BlockSpec((RB, C), lambda i: (i, 0)),
            pl.BlockSpec((RB, 1), lambda i: (i, 0)),
            pl.BlockSpec((RB, 1), lambda i: (i, 0)),
        ],
        out_shape=[
            jax.ShapeDtypeStruct((N, C), f32),
            jax.ShapeDtypeStruct((N, 1), f32),
            jax.ShapeDtypeStruct((N, 1), f32),
        ],
    )(x.astype(f32), W_src.T.astype(f32), avs.reshape(1, C), avd.reshape(1, C))

    EB = 16384
    sp, mp = pl.pallas_call(
        _prep_edges_body,
        grid=(EPAD // EB,),
        in_specs=[
            pl.BlockSpec((EB, ED), lambda i: (i, 0)),
            pl.BlockSpec((EB, MD), lambda i: (i, 0)),
            pl.BlockSpec((1, ED), lambda i: (0, 0)),
            pl.BlockSpec((1, MD), lambda i: (0, 0)),
        ],
        out_specs=[
            pl.BlockSpec((EB,), lambda i: (i,)),
            pl.BlockSpec((EB,), lambda i: (i,)),
        ],
        out_shape=[
            jax.ShapeDtypeStruct((EPAD,), f32),
            jax.ShapeDtypeStruct((EPAD,), f32),
        ],
    )(edge_attr.astype(f32), macro.astype(f32), ve.reshape(1, ED), vm.reshape(1, MD))

    npad = EPAD - E
    srcp = jnp.concatenate([edge_index[0], jnp.zeros((npad,), jnp.int32)])
    dstp = jnp.concatenate([edge_index[1], jnp.full((npad,), N, jnp.int32)])
    z32 = jnp.zeros((RPT, C), f32)
    z1 = jnp.zeros((RPT1,), f32)

    msg, den, sseg, deg = _sc_edge_pass(srcp, dstp, sp, mp,
                                        a_src[:, 0], a_dst[:, 0], xp, z32, z1)
    msg = msg.reshape(NC, NACC, C)
    den = den.reshape(NC, NACC1, 1)
    sseg = sseg.reshape(NC, NACC1, 1)
    deg = deg.reshape(NC, NACC1, 1)

    out = pl.pallas_call(
        _finish_body,
        grid=(N // RB,),
        in_specs=[
            pl.BlockSpec((NC, RB, C), lambda i: (0, i, 0)),
            pl.BlockSpec((NC, RB, 1), lambda i: (0, i, 0)),
            pl.BlockSpec((NC, RB, 1), lambda i: (0, i, 0)),
            pl.BlockSpec((NC, RB, 1), lambda i: (0, i, 0)),
            pl.BlockSpec((RB, C), lambda i: (i, 0)),
            pl.BlockSpec((RB, 1), lambda i: (i, 0)),
            pl.BlockSpec((RB, 1), lambda i: (i, 0)),
            pl.BlockSpec((1, C), lambda i: (0, 0)),
            pl.BlockSpec((1, 1), lambda i: (0, 0)),
        ],
        out_specs=pl.BlockSpec((RB, C), lambda i: (i, 0)),
        out_shape=jax.ShapeDtypeStruct((N, C), f32),
    )(msg, den, sseg, deg, xp, a_src, a_dst,
      bias.reshape(1, C).astype(f32), m0c)
    return out


# speedup vs baseline: 2.0370x; 1.0166x over previous
import functools

import jax
import jax.numpy as jnp
from jax import lax
from jax.experimental import pallas as pl
from jax.experimental.pallas import tpu as pltpu
from jax.experimental.pallas import tpu_sc as plsc

N = 50000
E = 800000
C = 32
FIN = 64
ED = 9
MD = 3

NC = 2
NS = 16
LANES = 16
NW = NC * NS

EK = 384
EPAD = ((E + NW * EK - 1) // (NW * EK)) * (NW * EK)
CHUNKS = EPAD // (NW * EK)
NACC = 51200
RPT = NACC // NS
ZC = 632
NZ = RPT // ZC
NACC1 = NACC
RPT1 = NACC1 // NS


def _prep_nodes_body(x_ref, wt_ref, avs_ref, avd_ref, xp_ref, as_ref, ad_ref):
    xp = jnp.dot(x_ref[...], wt_ref[...], preferred_element_type=jnp.float32)
    xp_ref[...] = xp
    as_ref[...] = jnp.sum(xp * avs_ref[...], axis=1, keepdims=True)
    ad_ref[...] = jnp.sum(xp * avd_ref[...], axis=1, keepdims=True)


def _prep_edges_body(ea_ref, mac_ref, ve_ref, vm_ref, s_ref, m_ref):
    s_ref[...] = jnp.sum(ea_ref[...] * ve_ref[...], axis=1)
    m_ref[...] = jnp.sum(mac_ref[...] * vm_ref[...], axis=1)


def _finish_body(msg_ref, den0_ref, den1_ref, sseg0_ref, sseg1_ref,
                 deg0_ref, deg1_ref, xp_ref, as_ref, ad_ref,
                 bias_ref, m0c_ref, out_ref):
    msg = msg_ref[0] + msg_ref[1]
    den = (den0_ref[...] + den1_ref[...]).reshape(-1, 1)
    sseg = (sseg0_ref[...] + sseg1_ref[...]).reshape(-1, 1)
    deg = (deg0_ref[...] + deg1_ref[...]).reshape(-1, 1)
    al = as_ref[...] + ad_ref[...] + sseg / jnp.maximum(deg, 1.0) + m0c_ref[0, 0]
    al = jnp.where(al >= 0.0, al, 0.2 * al)
    exl = jnp.exp(al)
    xp = xp_ref[...]
    out_ref[...] = (msg + xp * exl) / (den + exl + 1e-16) + bias_ref[...]


def _sc_body(src_r, dst_r, s_r, m_r, as_r, ad_r, xp_r, z32_r, z1_r,
             msg_out, den_out, sseg_out, deg_out,
             accm, accd, accs, accg,
             srcv, dstv, sv, mv, asv, adv, exb, oneb, msgv,
             sem0, sem1, sem2):
    cid = lax.axis_index("c")
    sid = lax.axis_index("s")
    wid = cid * NS + sid
    one16 = jnp.ones((LANES,), jnp.float32)

    row0 = sid * RPT
    srow0 = sid * RPT1
    pltpu.sync_copy(z32_r, accm.at[pl.ds(row0, RPT)])
    pltpu.sync_copy(z1_r, accd.at[pl.ds(srow0, RPT1)])
    pltpu.sync_copy(z1_r, accs.at[pl.ds(srow0, RPT1)])
    pltpu.sync_copy(z1_r, accg.at[pl.ds(srow0, RPT1)])

    for gi in range(EK // LANES):
        oneb[pl.ds(gi * LANES, LANES)] = one16

    plsc.subcore_barrier()

    ebase = wid * (CHUNKS * EK)

    def chunk(j, carry):
        base = ebase + j * EK
        pltpu.sync_copy(src_r.at[pl.ds(base, EK)], srcv)
        pltpu.sync_copy(dst_r.at[pl.ds(base, EK)], dstv)
        pltpu.sync_copy(s_r.at[pl.ds(base, EK)], sv)
        pltpu.sync_copy(m_r.at[pl.ds(base, EK)], mv)
        ca = pltpu.async_copy(as_r.at[srcv], asv, sem0)
        cb = pltpu.async_copy(ad_r.at[dstv], adv, sem1)
        cc = pltpu.async_copy(xp_r.at[srcv], msgv, sem2)
        ca.wait()
        cb.wait()
        cc.wait()
        for gi in range(EK // LANES):
            o = gi * LANES
            t = (asv[pl.ds(o, LANES)] + adv[pl.ds(o, LANES)]
                 + sv[pl.ds(o, LANES)] + mv[pl.ds(o, LANES)])
            t = jnp.where(t >= 0.0, t, 0.2 * t)
            exv = jnp.exp(t)
            exb[pl.ds(o, LANES)] = exv
            for l in range(LANES):
                e = o + l
                spl = exv.at[jnp.full((LANES,), l, jnp.int32)].get(
                    mode="promise_in_bounds")
                msgv[e, pl.ds(0, LANES)] = msgv[e, pl.ds(0, LANES)] * spl
                msgv[e, pl.ds(LANES, LANES)] = msgv[e, pl.ds(LANES, LANES)] * spl
        pltpu.sync_copy(msgv, accm.at[dstv], add=True)
        pltpu.sync_copy(exb, accd.at[dstv], add=True)
        pltpu.sync_copy(sv, accs.at[dstv], add=True)
        pltpu.sync_copy(oneb, accg.at[dstv], add=True)
        return carry

    lax.fori_loop(0, CHUNKS, chunk, 0)

    plsc.subcore_barrier()

    pltpu.sync_copy(accm.at[pl.ds(row0, RPT)],
                    msg_out.at[pl.ds(cid * NACC + row0, RPT)])
    pltpu.sync_copy(accd.at[pl.ds(srow0, RPT1)],
                    den_out.at[pl.ds(cid * NACC1 + srow0, RPT1)])
    pltpu.sync_copy(accs.at[pl.ds(srow0, RPT1)],
                    sseg_out.at[pl.ds(cid * NACC1 + srow0, RPT1)])
    pltpu.sync_copy(accg.at[pl.ds(srow0, RPT1)],
                    deg_out.at[pl.ds(cid * NACC1 + srow0, RPT1)])


_sc_edge_pass = functools.partial(
    pl.kernel,
    out_type=[
        jax.ShapeDtypeStruct((NC * NACC, C), jnp.float32),
        jax.ShapeDtypeStruct((NC * NACC1,), jnp.float32),
        jax.ShapeDtypeStruct((NC * NACC1,), jnp.float32),
        jax.ShapeDtypeStruct((NC * NACC1,), jnp.float32),
    ],
    mesh=plsc.VectorSubcoreMesh(core_axis_name="c", subcore_axis_name="s"),
    compiler_params=pltpu.CompilerParams(use_tc_tiling_on_sc=False),
    scratch_types=[
        pltpu.VMEM_SHARED((NACC, C), jnp.float32),
        pltpu.VMEM_SHARED((NACC1,), jnp.float32),
        pltpu.VMEM_SHARED((NACC1,), jnp.float32),
        pltpu.VMEM_SHARED((NACC1,), jnp.float32),
        pltpu.VMEM((EK,), jnp.int32),
        pltpu.VMEM((EK,), jnp.int32),
        pltpu.VMEM((EK,), jnp.float32),
        pltpu.VMEM((EK,), jnp.float32),
        pltpu.VMEM((EK,), jnp.float32),
        pltpu.VMEM((EK,), jnp.float32),
        pltpu.VMEM((EK,), jnp.float32),
        pltpu.VMEM((EK,), jnp.float32),
        pltpu.VMEM((EK, C), jnp.float32),
        pltpu.SemaphoreType.DMA,
        pltpu.SemaphoreType.DMA,
        pltpu.SemaphoreType.DMA,
    ],
)(_sc_body)


def kernel(x, edge_index, edge_attr, macro, W_src, att_src, att_dst,
           W_edge, att_edge, W_macro, att_macro, bias):
    f32 = jnp.float32
    avs = att_src[0, 0].astype(f32)
    avd = att_dst[0, 0].astype(f32)
    ve = (att_edge[0, 0] @ W_edge).astype(f32)
    vm = (att_macro[0, 0] @ W_macro).astype(f32)
    m0c = (macro[0] @ vm).reshape(1, 1).astype(f32)

    RB = 1000
    xp, a_src, a_dst = pl.pallas_call(
        _prep_nodes_body,
        grid=(N // RB,),
        in_specs=[
            pl.BlockSpec((RB, FIN), lambda i: (i, 0)),
            pl.BlockSpec((FIN, C), lambda i: (0, 0)),
            pl.BlockSpec((1, C), lambda i: (0, 0)),
            pl.BlockSpec((1, C), lambda i: (0, 0)),
        ],
        out_specs=[
            pl.BlockSpec((RB, C), lambda i: (i, 0)),
            pl.BlockSpec((RB, 1), lambda i: (i, 0)),
            pl.BlockSpec((RB, 1), lambda i: (i, 0)),
        ],
        out_shape=[
            jax.ShapeDtypeStruct((N, C), f32),
            jax.ShapeDtypeStruct((N, 1), f32),
            jax.ShapeDtypeStruct((N, 1), f32),
        ],
    )(x.astype(f32), W_src.T.astype(f32), avs.reshape(1, C), avd.reshape(1, C))

    EB = 16384
    sp, mp = pl.pallas_call(
        _prep_edges_body,
        grid=(EPAD // EB,),
        in_specs=[
            pl.BlockSpec((EB, ED), lambda i: (i, 0)),
            pl.BlockSpec((EB, MD), lambda i: (i, 0)),
            pl.BlockSpec((1, ED), lambda i: (0, 0)),
            pl.BlockSpec((1, MD), lambda i: (0, 0)),
        ],
        out_specs=[
            pl.BlockSpec((EB,), lambda i: (i,)),
            pl.BlockSpec((EB,), lambda i: (i,)),
        ],
        out_shape=[
            jax.ShapeDtypeStruct((EPAD,), f32),
            jax.ShapeDtypeStruct((EPAD,), f32),
        ],
    )(edge_attr.astype(f32), macro.astype(f32), ve.reshape(1, ED), vm.reshape(1, MD))

    npad = EPAD - E
    srcp = jnp.concatenate([edge_index[0], jnp.zeros((npad,), jnp.int32)])
    dstp = jnp.concatenate([edge_index[1], jnp.full((npad,), N, jnp.int32)])
    z32 = jnp.zeros((RPT, C), f32)
    z1 = jnp.zeros((RPT1,), f32)

    msg, den, sseg, deg = _sc_edge_pass(srcp, dstp, sp, mp,
                                        a_src[:, 0], a_dst[:, 0], xp, z32, z1)
    msg = msg.reshape(NC, NACC, C)

    FB = 512
    FG = NACC // FB
    xp_p = jnp.pad(xp, ((0, NACC - N), (0, 0)))
    as_p = jnp.pad(a_src, ((0, NACC - N), (0, 0)))
    ad_p = jnp.pad(a_dst, ((0, NACC - N), (0, 0)))
    out = pl.pallas_call(
        _finish_body,
        grid=(FG,),
        in_specs=[
            pl.BlockSpec((NC, FB, C), lambda i: (0, i, 0)),
            pl.BlockSpec((FB,), lambda i: (i,)),
            pl.BlockSpec((FB,), lambda i: (i + FG,)),
            pl.BlockSpec((FB,), lambda i: (i,)),
            pl.BlockSpec((FB,), lambda i: (i + FG,)),
            pl.BlockSpec((FB,), lambda i: (i,)),
            pl.BlockSpec((FB,), lambda i: (i + FG,)),
            pl.BlockSpec((FB, C), lambda i: (i, 0)),
            pl.BlockSpec((FB, 1), lambda i: (i, 0)),
            pl.BlockSpec((FB, 1), lambda i: (i, 0)),
            pl.BlockSpec((1, C), lambda i: (0, 0)),
            pl.BlockSpec((1, 1), lambda i: (0, 0)),
        ],
        out_specs=pl.BlockSpec((FB, C), lambda i: (i, 0)),
        out_shape=jax.ShapeDtypeStruct((NACC, C), f32),
    )(msg, den, den, sseg, sseg, deg, deg, xp_p, as_p, ad_p,
      bias.reshape(1, C).astype(f32), m0c)
    return out[:N]
